# baseline (speedup 1.0000x reference)
import jax
import jax.numpy as jnp
from jax import lax
from jax.experimental import pallas as pl
from jax.experimental.pallas import tpu as pltpu
from jax.experimental.pallas import tpu_sc as plsc

N = 10000
E = 320000
D = 128
L = 4
NC = 2
NS = 16
NW = NC * NS
CHUNK = 128
CPT = 80
EPAD = NW * CPT * CHUNK
NAGG = 10240
RPT = NAGG // NS
RB = RPT // CHUNK

ACH = 128
ACPT = 80
AEPAD = NW * ACPT * ACH
ANAGG = 10240
ARPT = ANAGG // NS
ANST = ACPT // 4

_HIGH = jax.lax.Precision.HIGHEST


def _sc_deg_body(srcd, dstp, ones_h, zeros_h, outs, outd, sidx, ones_v, zv,
                 deg_sh):
    c = lax.axis_index("c")
    s = lax.axis_index("s")
    wid = s * NC + c
    ebase = wid * CPT * CHUNK
    pltpu.sync_copy(ones_h, ones_v)
    pltpu.sync_copy(zeros_h, zv)
    base = s * RPT
    for kz in range(RB):
        pltpu.sync_copy(zv, deg_sh.at[pl.ds(base + kz * CHUNK, CHUNK)])
    plsc.subcore_barrier()

    def mkstep(idx_hbm):
        def step(j, carry):
            pltpu.sync_copy(idx_hbm.at[pl.ds(ebase + j * CHUNK, CHUNK)],
                            sidx.at[0])
            pltpu.sync_copy(ones_v, deg_sh.at[sidx.at[0]], add=True)
            return carry
        return step

    lax.fori_loop(0, CPT, mkstep(dstp), 0)
    plsc.subcore_barrier()
    for ko in range(RB):
        r0 = base + ko * CHUNK
        pltpu.sync_copy(deg_sh.at[pl.ds(r0, CHUNK)],
                        outd.at[c].at[pl.ds(r0, CHUNK)])
        pltpu.sync_copy(zv, deg_sh.at[pl.ds(r0, CHUNK)])
    plsc.subcore_barrier()
    lax.fori_loop(0, CPT, mkstep(srcd), 0)
    plsc.subcore_barrier()
    for ko in range(RB):
        r0 = base + ko * CHUNK
        pltpu.sync_copy(deg_sh.at[pl.ds(r0, CHUNK)],
                        outs.at[c].at[pl.ds(r0, CHUNK)])


def _sc_agg_body(srcg, dstp, xs, zeros_h, out, sidx, didx, rows0, rows1,
                 i0, i1, i2, i3, g0, g1, s0, s1, agg_sh):
    c = lax.axis_index("c")
    s = lax.axis_index("s")
    wid = s * NC + c
    ebase = wid * ACPT * ACH
    isems = (i0, i1, i2, i3)

    def load_idx(b, j):
        off = ebase + j * ACH
        pltpu.async_copy(srcg.at[pl.ds(off, ACH)], sidx.at[b], isems[b])
        pltpu.async_copy(dstp.at[pl.ds(off, ACH)], didx.at[b], isems[b])

    def wait_idx(b):
        pltpu.make_async_copy(srcg.at[pl.ds(ebase, ACH)], sidx.at[b],
                              isems[b]).wait()
        pltpu.make_async_copy(dstp.at[pl.ds(ebase, ACH)], didx.at[b],
                              isems[b]).wait()

    for b in range(4):
        load_idx(b, b)

    pltpu.sync_copy(zeros_h, rows0)
    base = s * ARPT
    for kz in range(5):
        pltpu.sync_copy(rows0, agg_sh.at[pl.ds(base + kz * ACH, ACH)])
    plsc.subcore_barrier()

    def reload_idx(b, j, t):
        @pl.when(t + 1 < ANST)
        def _():
            load_idx(b, j)

    def step(t, carry):
        c0 = 4 * t
        wait_idx(0)
        pltpu.async_copy(xs.at[sidx.at[0]], rows0, g0)
        wait_idx(1)
        pltpu.async_copy(xs.at[sidx.at[1]], rows1, g1)
        pltpu.make_async_copy(xs.at[sidx.at[0]], rows0, g0).wait()
        pltpu.async_copy(rows0, agg_sh.at[didx.at[0]], s0, add=True)
        pltpu.make_async_copy(xs.at[sidx.at[1]], rows1, g1).wait()
        pltpu.async_copy(rows1, agg_sh.at[didx.at[1]], s1, add=True)
        pltpu.make_async_copy(rows0, agg_sh.at[didx.at[0]], s0).wait()
        reload_idx(0, c0 + 4, t)
        wait_idx(2)
        pltpu.async_copy(xs.at[sidx.at[2]], rows0, g0)
        pltpu.make_async_copy(rows1, agg_sh.at[didx.at[1]], s1).wait()
        reload_idx(1, c0 + 5, t)
        wait_idx(3)
        pltpu.async_copy(xs.at[sidx.at[3]], rows1, g1)
        pltpu.make_async_copy(xs.at[sidx.at[2]], rows0, g0).wait()
        pltpu.async_copy(rows0, agg_sh.at[didx.at[2]], s0, add=True)
        pltpu.make_async_copy(xs.at[sidx.at[3]], rows1, g1).wait()
        pltpu.async_copy(rows1, agg_sh.at[didx.at[3]], s1, add=True)
        pltpu.make_async_copy(rows0, agg_sh.at[didx.at[2]], s0).wait()
        reload_idx(2, c0 + 6, t)
        pltpu.make_async_copy(rows1, agg_sh.at[didx.at[3]], s1).wait()
        reload_idx(3, c0 + 7, t)
        return carry

    lax.fori_loop(0, ANST, step, 0)
    plsc.subcore_barrier()
    for ko in range(5):
        r0 = base + ko * ACH
        pltpu.sync_copy(agg_sh.at[pl.ds(r0, ACH)],
                        out.at[c].at[pl.ds(r0, ACH)])


def _make_sc_deg():
    mesh = plsc.VectorSubcoreMesh(core_axis_name="c", subcore_axis_name="s")
    return pl.kernel(
        _sc_deg_body,
        out_type=[jax.ShapeDtypeStruct((NC, NAGG, D), jnp.float32),
                  jax.ShapeDtypeStruct((NC, NAGG, D), jnp.float32)],
        mesh=mesh,
        scratch_types=[
            pltpu.VMEM((1, CHUNK), jnp.int32),
            pltpu.VMEM((CHUNK, D), jnp.float32),
            pltpu.VMEM((CHUNK, D), jnp.float32),
            pltpu.VMEM_SHARED((NAGG, D), jnp.float32),
        ],
    )


def _make_sc_agg():
    mesh = plsc.VectorSubcoreMesh(core_axis_name="c", subcore_axis_name="s")
    return pl.kernel(
        _sc_agg_body,
        out_type=jax.ShapeDtypeStruct((NC, ANAGG, D), jnp.float32),
        mesh=mesh,
        scratch_types=[
            pltpu.VMEM((4, ACH), jnp.int32),
            pltpu.VMEM((4, ACH), jnp.int32),
            pltpu.VMEM((ACH, D), jnp.float32),
            pltpu.VMEM((ACH, D), jnp.float32),
        ] + [pltpu.SemaphoreType.DMA] * 8 + [
            pltpu.VMEM_SHARED((ANAGG, D), jnp.float32),
        ],
    )


def _tc_norms_body(degs_ref, degd_ref, ns_ref, nd_ref):
    ds_ = degs_ref[0] + degs_ref[1]
    dd = degd_ref[0] + degd_ref[1]
    ns_ref[...] = 1.0 / jnp.sqrt(jnp.maximum(ds_, 1.0))
    nd_ref[...] = 1.0 / jnp.sqrt(jnp.maximum(dd, 1.0))


def _tc_pre0_body(h_ref, emb_ref, ns_ref, x_ref, xs_ref):
    oh = (h_ref[...] == lax.broadcasted_iota(jnp.int32, (N, D), 1))
    x = jnp.dot(oh.astype(jnp.float32), emb_ref[...], precision=_HIGH,
                preferred_element_type=jnp.float32)
    x_ref[...] = x
    xs_ref[...] = x * ns_ref[...]


def _tc_mm_body(aggp_ref, nd_ref, w_ref, b_ref, t_ref, mu_ref, var_ref):
    agg = (aggp_ref[0, :N, :] + aggp_ref[1, :N, :]) * nd_ref[...]
    t = jnp.dot(agg, w_ref[...], precision=_HIGH,
                preferred_element_type=jnp.float32) + b_ref[...]
    t_ref[...] = t
    mu = jnp.mean(t, axis=0, keepdims=True)
    mu_ref[...] = mu
    var_ref[...] = jnp.mean((t - mu) ** 2, axis=0, keepdims=True)


def _bn_relu_res(t_ref, mu_ref, var_ref, g_ref, be_ref, x_ref):
    t = (t_ref[...] - mu_ref[...]) / jnp.sqrt(var_ref[...] + 1e-5)
    t = t * g_ref[...] + be_ref[...]
    return x_ref[...] + jnp.maximum(t, 0.0)


def _tc_post_body(t_ref, mu_ref, var_ref, g_ref, be_ref, x_ref,
                  ns_ref, xn_ref, xsn_ref):
    xn = _bn_relu_res(t_ref, mu_ref, var_ref, g_ref, be_ref, x_ref)
    xn_ref[...] = xn
    xsn_ref[...] = xn * ns_ref[...]


def _tc_final_body(t_ref, mu_ref, var_ref, g_ref, be_ref, x_ref,
                   w0_ref, b0_ref, w1_ref, b1_ref, w2_ref, b2_ref, out_ref):
    xn = _bn_relu_res(t_ref, mu_ref, var_ref, g_ref, be_ref, x_ref)
    y = jnp.maximum(jnp.dot(xn, w0_ref[...], precision=_HIGH,
                            preferred_element_type=jnp.float32)
                    + b0_ref[...], 0.0)
    y = jnp.maximum(jnp.dot(y, w1_ref[...], precision=_HIGH,
                            preferred_element_type=jnp.float32)
                    + b1_ref[...], 0.0)
    out_ref[...] = jnp.dot(y, w2_ref[...], precision=_HIGH,
                           preferred_element_type=jnp.float32) + b2_ref[...]


def _tc_call(body, out_shapes):
    return pl.pallas_call(
        body,
        out_shape=out_shapes,
    )


def kernel(h, edge_index, emb, Ws, bs, gammas, betas, W0, b0, W1, b1, W2, b2):
    src = edge_index[0].astype(jnp.int32)
    dst = edge_index[1].astype(jnp.int32)
    pad = EPAD - E
    pad_a = AEPAD - E
    src_d = jnp.concatenate([src, jnp.full((pad,), N, jnp.int32)])
    dst_p = jnp.concatenate([dst, jnp.full((pad,), N, jnp.int32)])
    src_g = jnp.concatenate([src, jnp.zeros((pad_a,), jnp.int32)])
    dst_g = jnp.concatenate([dst, jnp.full((pad_a,), N, jnp.int32)])

    ones_h = jnp.ones((CHUNK, D), jnp.float32)
    zeros_h = jnp.zeros((CHUNK, D), jnp.float32)

    degs, degd = _make_sc_deg()(src_d, dst_p, ones_h, zeros_h)
    norms_s, norms_d = _tc_call(
        _tc_norms_body,
        [jax.ShapeDtypeStruct((NAGG, D), jnp.float32),
         jax.ShapeDtypeStruct((NAGG, D), jnp.float32)])(degs, degd)
    ns_col = norms_s[:N, 0:1]
    nd_col = norms_d[:N, 0:1]

    h2 = h.astype(jnp.int32).reshape(N, 1)
    x, xs = _tc_call(_tc_pre0_body,
                     [jax.ShapeDtypeStruct((N, D), jnp.float32),
                      jax.ShapeDtypeStruct((N, D), jnp.float32)])(
        h2, emb, ns_col)

    sc_agg = _make_sc_agg()
    for i in range(L):
        aggp = sc_agg(src_g, dst_g, xs, zeros_h)
        w = Ws[i]
        b = bs[i].reshape(1, D)
        g = gammas[i].reshape(1, D)
        be = betas[i].reshape(1, D)
        t, mu, var = _tc_call(_tc_mm_body,
                              [jax.ShapeDtypeStruct((N, D), jnp.float32),
                               jax.ShapeDtypeStruct((1, D), jnp.float32),
                               jax.ShapeDtypeStruct((1, D), jnp.float32)])(
            aggp, nd_col, w, b)
        if i < L - 1:
            x, xs = _tc_call(_tc_post_body,
                             [jax.ShapeDtypeStruct((N, D), jnp.float32),
                              jax.ShapeDtypeStruct((N, D), jnp.float32)])(
                t, mu, var, g, be, x, ns_col)
        else:
            out = _tc_call(_tc_final_body,
                           jax.ShapeDtypeStruct((N, D), jnp.float32))(
                t, mu, var, g, be, x,
                W0, b0.reshape(1, D // 2), W1, b1.reshape(1, D // 4),
                W2, b2.reshape(1, D))
    return out


# speedup vs baseline: 1.0087x; 1.0087x over previous
import jax
import jax.numpy as jnp
from jax import lax
from jax.experimental import pallas as pl
from jax.experimental.pallas import tpu as pltpu
from jax.experimental.pallas import tpu_sc as plsc

N = 10000
E = 320000
D = 128
L = 4
NC = 2
NS = 16
NW = NC * NS
CHUNK = 128
CPT = 80
EPAD = NW * CPT * CHUNK
NAGG = 10240
RPT = NAGG // NS
RB = RPT // CHUNK

ACH = 128
ACPT = 80
AEPAD = NW * ACPT * ACH
ANAGG = 10240
ARPT = ANAGG // NS
ANST = ACPT // 4

_HIGH = jax.lax.Precision.HIGHEST


def _dot3(a, w):
    ah = a.astype(jnp.bfloat16)
    wh = w.astype(jnp.bfloat16)
    return jnp.dot(ah, wh, preferred_element_type=jnp.float32)


def _sc_deg_body(srcd, dstp, ones_h, zeros_h, outs, outd, sidx, ones_v, zv,
                 deg_sh):
    c = lax.axis_index("c")
    s = lax.axis_index("s")
    wid = s * NC + c
    ebase = wid * CPT * CHUNK
    pltpu.sync_copy(ones_h, ones_v)
    pltpu.sync_copy(zeros_h, zv)
    base = s * RPT
    for kz in range(RB):
        pltpu.sync_copy(zv, deg_sh.at[pl.ds(base + kz * CHUNK, CHUNK)])
    plsc.subcore_barrier()

    def mkstep(idx_hbm):
        def step(j, carry):
            pltpu.sync_copy(idx_hbm.at[pl.ds(ebase + j * CHUNK, CHUNK)],
                            sidx.at[0])
            pltpu.sync_copy(ones_v, deg_sh.at[sidx.at[0]], add=True)
            return carry
        return step

    lax.fori_loop(0, CPT, mkstep(dstp), 0)
    plsc.subcore_barrier()
    for ko in range(RB):
        r0 = base + ko * CHUNK
        pltpu.sync_copy(deg_sh.at[pl.ds(r0, CHUNK)],
                        outd.at[c].at[pl.ds(r0, CHUNK)])
        pltpu.sync_copy(zv, deg_sh.at[pl.ds(r0, CHUNK)])
    plsc.subcore_barrier()
    lax.fori_loop(0, CPT, mkstep(srcd), 0)
    plsc.subcore_barrier()
    for ko in range(RB):
        r0 = base + ko * CHUNK
        pltpu.sync_copy(deg_sh.at[pl.ds(r0, CHUNK)],
                        outs.at[c].at[pl.ds(r0, CHUNK)])


def _sc_agg_body(srcg, dstp, xs, zeros_h, out, sidx, didx, rows0, rows1,
                 i0, i1, i2, i3, g0, g1, s0, s1, agg_sh):
    c = lax.axis_index("c")
    s = lax.axis_index("s")
    wid = s * NC + c
    ebase = wid * ACPT * ACH
    isems = (i0, i1, i2, i3)

    def load_idx(b, j):
        off = ebase + j * ACH
        pltpu.async_copy(srcg.at[pl.ds(off, ACH)], sidx.at[b], isems[b])
        pltpu.async_copy(dstp.at[pl.ds(off, ACH)], didx.at[b], isems[b])

    def wait_idx(b):
        pltpu.make_async_copy(srcg.at[pl.ds(ebase, ACH)], sidx.at[b],
                              isems[b]).wait()
        pltpu.make_async_copy(dstp.at[pl.ds(ebase, ACH)], didx.at[b],
                              isems[b]).wait()

    for b in range(4):
        load_idx(b, b)

    pltpu.sync_copy(zeros_h, rows0)
    base = s * ARPT
    for kz in range(5):
        pltpu.sync_copy(rows0, agg_sh.at[pl.ds(base + kz * ACH, ACH)])
    plsc.subcore_barrier()

    def reload_idx(b, j, t):
        @pl.when(t + 1 < ANST)
        def _():
            load_idx(b, j)

    def step(t, carry):
        c0 = 4 * t
        wait_idx(0)
        pltpu.async_copy(xs.at[sidx.at[0]], rows0, g0)
        wait_idx(1)
        pltpu.async_copy(xs.at[sidx.at[1]], rows1, g1)
        pltpu.make_async_copy(xs.at[sidx.at[0]], rows0, g0).wait()
        pltpu.async_copy(rows0, agg_sh.at[didx.at[0]], s0, add=True)
        pltpu.make_async_copy(xs.at[sidx.at[1]], rows1, g1).wait()
        pltpu.async_copy(rows1, agg_sh.at[didx.at[1]], s1, add=True)
        pltpu.make_async_copy(rows0, agg_sh.at[didx.at[0]], s0).wait()
        reload_idx(0, c0 + 4, t)
        wait_idx(2)
        pltpu.async_copy(xs.at[sidx.at[2]], rows0, g0)
        pltpu.make_async_copy(rows1, agg_sh.at[didx.at[1]], s1).wait()
        reload_idx(1, c0 + 5, t)
        wait_idx(3)
        pltpu.async_copy(xs.at[sidx.at[3]], rows1, g1)
        pltpu.make_async_copy(xs.at[sidx.at[2]], rows0, g0).wait()
        pltpu.async_copy(rows0, agg_sh.at[didx.at[2]], s0, add=True)
        pltpu.make_async_copy(xs.at[sidx.at[3]], rows1, g1).wait()
        pltpu.async_copy(rows1, agg_sh.at[didx.at[3]], s1, add=True)
        pltpu.make_async_copy(rows0, agg_sh.at[didx.at[2]], s0).wait()
        reload_idx(2, c0 + 6, t)
        pltpu.make_async_copy(rows1, agg_sh.at[didx.at[3]], s1).wait()
        reload_idx(3, c0 + 7, t)
        return carry

    lax.fori_loop(0, ANST, step, 0)
    plsc.subcore_barrier()
    for ko in range(5):
        r0 = base + ko * ACH
        pltpu.sync_copy(agg_sh.at[pl.ds(r0, ACH)],
                        out.at[c].at[pl.ds(r0, ACH)])


def _make_sc_deg():
    mesh = plsc.VectorSubcoreMesh(core_axis_name="c", subcore_axis_name="s")
    return pl.kernel(
        _sc_deg_body,
        out_type=[jax.ShapeDtypeStruct((NC, NAGG, D), jnp.float32),
                  jax.ShapeDtypeStruct((NC, NAGG, D), jnp.float32)],
        mesh=mesh,
        scratch_types=[
            pltpu.VMEM((1, CHUNK), jnp.int32),
            pltpu.VMEM((CHUNK, D), jnp.float32),
            pltpu.VMEM((CHUNK, D), jnp.float32),
            pltpu.VMEM_SHARED((NAGG, D), jnp.float32),
        ],
    )


def _make_sc_agg():
    mesh = plsc.VectorSubcoreMesh(core_axis_name="c", subcore_axis_name="s")
    return pl.kernel(
        _sc_agg_body,
        out_type=jax.ShapeDtypeStruct((NC, ANAGG, D), jnp.float32),
        mesh=mesh,
        scratch_types=[
            pltpu.VMEM((4, ACH), jnp.int32),
            pltpu.VMEM((4, ACH), jnp.int32),
            pltpu.VMEM((ACH, D), jnp.float32),
            pltpu.VMEM((ACH, D), jnp.float32),
        ] + [pltpu.SemaphoreType.DMA] * 8 + [
            pltpu.VMEM_SHARED((ANAGG, D), jnp.float32),
        ],
    )


def _tc_norms_body(degs_ref, degd_ref, ns_ref, nd_ref):
    ds_ = degs_ref[0] + degs_ref[1]
    dd = degd_ref[0] + degd_ref[1]
    ns_ref[...] = 1.0 / jnp.sqrt(jnp.maximum(ds_, 1.0))
    nd_ref[...] = 1.0 / jnp.sqrt(jnp.maximum(dd, 1.0))


def _tc_pre0_body(h_ref, emb_ref, ns_ref, x_ref, xs_ref):
    oh = (h_ref[...] == lax.broadcasted_iota(jnp.int32, (N, D), 1))
    x = jnp.dot(oh.astype(jnp.float32), emb_ref[...], precision=_HIGH,
                preferred_element_type=jnp.float32)
    x_ref[...] = x
    xs_ref[...] = x * ns_ref[...]


def _tc_mm_body(aggp_ref, nd_ref, w_ref, b_ref, t_ref, mu_ref, var_ref):
    agg = (aggp_ref[0, :N, :] + aggp_ref[1, :N, :]) * nd_ref[...]
    t = _dot3(agg, w_ref[...]) + b_ref[...]
    t_ref[...] = t
    mu = jnp.mean(t, axis=0, keepdims=True)
    mu_ref[...] = mu
    var_ref[...] = jnp.mean((t - mu) ** 2, axis=0, keepdims=True)


def _bn_relu_res(t_ref, mu_ref, var_ref, g_ref, be_ref, x_ref):
    t = (t_ref[...] - mu_ref[...]) / jnp.sqrt(var_ref[...] + 1e-5)
    t = t * g_ref[...] + be_ref[...]
    return x_ref[...] + jnp.maximum(t, 0.0)


def _tc_post_body(t_ref, mu_ref, var_ref, g_ref, be_ref, x_ref,
                  ns_ref, xn_ref, xsn_ref):
    xn = _bn_relu_res(t_ref, mu_ref, var_ref, g_ref, be_ref, x_ref)
    xn_ref[...] = xn
    xsn_ref[...] = xn * ns_ref[...]


def _tc_final_body(t_ref, mu_ref, var_ref, g_ref, be_ref, x_ref,
                   w0_ref, b0_ref, w1_ref, b1_ref, w2_ref, b2_ref, out_ref):
    xn = _bn_relu_res(t_ref, mu_ref, var_ref, g_ref, be_ref, x_ref)
    y = jnp.maximum(_dot3(xn, w0_ref[...]) + b0_ref[...], 0.0)
    y = jnp.maximum(_dot3(y, w1_ref[...]) + b1_ref[...], 0.0)
    out_ref[...] = _dot3(y, w2_ref[...]) + b2_ref[...]


def _tc_call(body, out_shapes):
    return pl.pallas_call(
        body,
        out_shape=out_shapes,
    )


def kernel(h, edge_index, emb, Ws, bs, gammas, betas, W0, b0, W1, b1, W2, b2):
    src = edge_index[0].astype(jnp.int32)
    dst = edge_index[1].astype(jnp.int32)
    pad = EPAD - E
    pad_a = AEPAD - E
    src_d = jnp.concatenate([src, jnp.full((pad,), N, jnp.int32)])
    dst_p = jnp.concatenate([dst, jnp.full((pad,), N, jnp.int32)])
    src_g = jnp.concatenate([src, jnp.zeros((pad_a,), jnp.int32)])
    dst_g = jnp.concatenate([dst, jnp.full((pad_a,), N, jnp.int32)])

    ones_h = jnp.ones((CHUNK, D), jnp.float32)
    zeros_h = jnp.zeros((CHUNK, D), jnp.float32)

    degs, degd = _make_sc_deg()(src_d, dst_p, ones_h, zeros_h)
    norms_s, norms_d = _tc_call(
        _tc_norms_body,
        [jax.ShapeDtypeStruct((NAGG, D), jnp.float32),
         jax.ShapeDtypeStruct((NAGG, D), jnp.float32)])(degs, degd)
    ns_col = norms_s[:N, 0:1]
    nd_col = norms_d[:N, 0:1]

    h2 = h.astype(jnp.int32).reshape(N, 1)
    x, xs = _tc_call(_tc_pre0_body,
                     [jax.ShapeDtypeStruct((N, D), jnp.float32),
                      jax.ShapeDtypeStruct((N, D), jnp.float32)])(
        h2, emb, ns_col)

    sc_agg = _make_sc_agg()
    for i in range(L):
        aggp = sc_agg(src_g, dst_g, xs, zeros_h)
        w = Ws[i]
        b = bs[i].reshape(1, D)
        g = gammas[i].reshape(1, D)
        be = betas[i].reshape(1, D)
        t, mu, var = _tc_call(_tc_mm_body,
                              [jax.ShapeDtypeStruct((N, D), jnp.float32),
                               jax.ShapeDtypeStruct((1, D), jnp.float32),
                               jax.ShapeDtypeStruct((1, D), jnp.float32)])(
            aggp, nd_col, w, b)
        if i < L - 1:
            x, xs = _tc_call(_tc_post_body,
                             [jax.ShapeDtypeStruct((N, D), jnp.float32),
                              jax.ShapeDtypeStruct((N, D), jnp.float32)])(
                t, mu, var, g, be, x, ns_col)
        else:
            out = _tc_call(_tc_final_body,
                           jax.ShapeDtypeStruct((N, D), jnp.float32))(
                t, mu, var, g, be, x,
                W0, b0.reshape(1, D // 2), W1, b1.reshape(1, D // 4),
                W2, b2.reshape(1, D))
    return out
